# baseline (speedup 1.0000x reference)
import functools

import jax
import jax.numpy as jnp
from jax import lax
from jax.experimental import pallas as pl
from jax.experimental.pallas import tpu as pltpu
from jax.experimental.pallas import tpu_sc as plsc

NUM_EMB = 1000000
D = 64
B = 4096 * 200
NC, NS, L = 2, 16, 16
NW = NC * NS
NPW = B // NW
SUB = 128
CH = 512
NSUB = CH // SUB
NCHUNK = NPW // CH


def _emb_body(idx_hbm, table_hbm, out_hbm, idx_v, rows_v, gsem):
    wid = lax.axis_index("s") * NC + lax.axis_index("c")
    base = wid * NPW

    pltpu.sync_copy(idx_hbm.at[pl.ds(base, NPW)], idx_v)

    def remap(j, carry):
        off = pl.multiple_of(j * L, L)
        v = idx_v[pl.ds(off, L)]
        idx_v[pl.ds(off, L)] = jnp.where(v < 0, v + (NUM_EMB + 1), v)
        return carry

    lax.fori_loop(0, NPW // L, remap, 0)

    def chunk(c, carry):
        off = pl.multiple_of(c * CH, CH)
        waits = []
        for s in range(NSUB):
            waits.append(
                pltpu.async_copy(
                    table_hbm.at[idx_v.at[pl.ds(off + s * SUB, SUB)]],
                    rows_v.at[pl.ds(s * SUB, SUB)],
                    gsem,
                )
            )
        for w in waits:
            w.wait()
        pltpu.sync_copy(rows_v, out_hbm.at[pl.ds(base + off, CH)])
        return carry

    lax.fori_loop(0, NCHUNK, chunk, 0)


@jax.jit
def kernel(tgt, table):
    idx = tgt.reshape(-1).astype(jnp.int32)
    run = pl.kernel(
        _emb_body,
        out_type=jax.ShapeDtypeStruct((B, D), jnp.float32),
        mesh=plsc.VectorSubcoreMesh(
            core_axis_name="c", subcore_axis_name="s",
            num_cores=NC, num_subcores=NS,
        ),
        scratch_types=[
            pltpu.VMEM((NPW,), jnp.int32),
            pltpu.VMEM((CH, D), jnp.float32),
            pltpu.SemaphoreType.DMA,
        ],
        compiler_params=pltpu.CompilerParams(use_tc_tiling_on_sc=False),
    )
    out = run(idx, table)
    return out.reshape(tgt.shape + (D,))


# speedup vs baseline: 1.0255x; 1.0255x over previous
import jax
import jax.numpy as jnp
from jax import lax
from jax.experimental import pallas as pl
from jax.experimental.pallas import tpu as pltpu
from jax.experimental.pallas import tpu_sc as plsc

NUM_EMB = 1000000
D = 64
B = 4096 * 200
NC, NS, L = 2, 16, 16
NW = NC * NS
NPW = B // NW
SUB = 128
CH = 512
NSUB = CH // SUB
NCHUNK = NPW // CH
NBUF = 3


def _emb_body(idx_hbm, table_hbm, out_hbm,
              idx_v, rows0, rows1, rows2,
              gsem0, gsem1, gsem2, osem0, osem1, osem2):
    wid = lax.axis_index("s") * NC + lax.axis_index("c")
    base = wid * NPW
    rows = (rows0, rows1, rows2)
    gsem = (gsem0, gsem1, gsem2)
    osem = (osem0, osem1, osem2)

    pltpu.sync_copy(idx_hbm.at[pl.ds(base, NPW)], idx_v)

    def remap_chunk(c):
        off = pl.multiple_of(c * CH, CH)
        for j in range(CH // L):
            v = idx_v[pl.ds(off + j * L, L)]
            idx_v[pl.ds(off + j * L, L)] = jnp.where(v < 0, v + (NUM_EMB + 1), v)

    def gather_descs(c, b):
        off = pl.multiple_of(c * CH, CH)
        return [
            pltpu.make_async_copy(
                table_hbm.at[idx_v.at[pl.ds(off + s * SUB, SUB)]],
                rows[b].at[pl.ds(s * SUB, SUB)],
                gsem[b],
            )
            for s in range(NSUB)
        ]

    def issue_gather(c, b):
        remap_chunk(c)
        for d in gather_descs(c, b):
            d.start()

    def wait_gather(c, b):
        for d in gather_descs(c, b):
            d.wait()

    def write_desc(c, b):
        off = pl.multiple_of(c * CH, CH)
        return pltpu.make_async_copy(
            rows[b], out_hbm.at[pl.ds(base + off, CH)], osem[b])

    issue_gather(0, 0)
    for j in (0, 1):
        issue_gather(j + 1, j + 1)
        wait_gather(j, j)
        write_desc(j, j).start()

    def step(k, carry):
        for m in range(NBUF):
            j = k * NBUF + 2 + m
            b = (2 + m) % NBUF
            write_desc(j - 2, (b + 1) % NBUF).wait()
            issue_gather(j + 1, (b + 1) % NBUF)
            wait_gather(j, b)
            write_desc(j, b).start()
        return carry

    lax.fori_loop(0, (NCHUNK - 5) // NBUF, step, 0)

    for j in (NCHUNK - 3, NCHUNK - 2):
        b = j % NBUF
        write_desc(j - 2, (b + 1) % NBUF).wait()
        issue_gather(j + 1, (b + 1) % NBUF)
        wait_gather(j, b)
        write_desc(j, b).start()
    j = NCHUNK - 1
    b = j % NBUF
    write_desc(j - 2, (b + 1) % NBUF).wait()
    wait_gather(j, b)
    write_desc(j, b).start()
    for j in (NCHUNK - 2, NCHUNK - 1):
        write_desc(j, j % NBUF).wait()


@jax.jit
def kernel(tgt, table):
    idx = tgt.reshape(-1).astype(jnp.int32)
    run = pl.kernel(
        _emb_body,
        out_type=jax.ShapeDtypeStruct((B, D), jnp.float32),
        mesh=plsc.VectorSubcoreMesh(
            core_axis_name="c", subcore_axis_name="s",
            num_cores=NC, num_subcores=NS,
        ),
        scratch_types=[
            pltpu.VMEM((NPW,), jnp.int32),
            pltpu.VMEM((CH, D), jnp.float32),
            pltpu.VMEM((CH, D), jnp.float32),
            pltpu.VMEM((CH, D), jnp.float32),
            pltpu.SemaphoreType.DMA,
            pltpu.SemaphoreType.DMA,
            pltpu.SemaphoreType.DMA,
            pltpu.SemaphoreType.DMA,
            pltpu.SemaphoreType.DMA,
            pltpu.SemaphoreType.DMA,
        ],
        compiler_params=pltpu.CompilerParams(use_tc_tiling_on_sc=False),
    )
    out = run(idx, table)
    return out.reshape(tgt.shape + (D,))


# speedup vs baseline: 1.3619x; 1.3280x over previous
import jax
import jax.numpy as jnp
from jax import lax
from jax.experimental import pallas as pl
from jax.experimental.pallas import tpu as pltpu
from jax.experimental.pallas import tpu_sc as plsc

NUM_EMB = 1000000
D = 64
B = 4096 * 200
NC, NS, L = 2, 16, 16
NW = NC * NS
NPW = B // NW
SUB = 128
CH = 512
NSUB = CH // SUB
NCHUNK = NPW // CH
NBUF = 3


def _emb_body(idx_hbm, table_hbm, out_hbm,
              idx_v, rows0, rows1, rows2,
              gsem0, gsem1, gsem2, osem0, osem1, osem2):
    wid = lax.axis_index("s") * NC + lax.axis_index("c")
    base = wid * NPW
    rows = (rows0, rows1, rows2)
    gsem = (gsem0, gsem1, gsem2)
    osem = (osem0, osem1, osem2)

    pltpu.sync_copy(idx_hbm.at[pl.ds(base, NPW)], idx_v)

    def remap_chunk(c):
        off = pl.multiple_of(c * CH, CH)
        for j in range(CH // L):
            v = idx_v[pl.ds(off + j * L, L)]
            idx_v[pl.ds(off + j * L, L)] = jnp.where(v < 0, v + (NUM_EMB + 1), v)

    def gather_descs(c, b):
        off = pl.multiple_of(c * CH, CH)
        return [
            pltpu.make_async_copy(
                table_hbm.at[idx_v.at[pl.ds(off + s * SUB, SUB)]],
                rows[b].at[pl.ds(s * SUB, SUB)],
                gsem[b],
            )
            for s in range(NSUB)
        ]

    def issue_gather(c, b):
        remap_chunk(c)
        for d in gather_descs(c, b):
            d.start()

    def wait_gather(c, b):
        for d in gather_descs(c, b):
            d.wait()

    def write_desc(c, b):
        off = pl.multiple_of(c * CH, CH)
        return pltpu.make_async_copy(
            rows[b], out_hbm.at[pl.ds(base + off, CH), pl.ds(0, D)], osem[b])

    issue_gather(0, 0)
    for j in (0, 1):
        issue_gather(j + 1, j + 1)
        wait_gather(j, j)
        write_desc(j, j).start()

    def step(k, carry):
        for m in range(NBUF):
            j = k * NBUF + 2 + m
            b = (2 + m) % NBUF
            write_desc(j - 2, (b + 1) % NBUF).wait()
            issue_gather(j + 1, (b + 1) % NBUF)
            wait_gather(j, b)
            write_desc(j, b).start()
        return carry

    lax.fori_loop(0, (NCHUNK - 5) // NBUF, step, 0)

    for j in (NCHUNK - 3, NCHUNK - 2):
        b = j % NBUF
        write_desc(j - 2, (b + 1) % NBUF).wait()
        issue_gather(j + 1, (b + 1) % NBUF)
        wait_gather(j, b)
        write_desc(j, b).start()
    j = NCHUNK - 1
    b = j % NBUF
    write_desc(j - 2, (b + 1) % NBUF).wait()
    wait_gather(j, b)
    write_desc(j, b).start()
    for j in (NCHUNK - 2, NCHUNK - 1):
        write_desc(j, j % NBUF).wait()


@jax.jit
def kernel(tgt, table):
    idx = tgt.reshape(-1).astype(jnp.int32)
    run = pl.kernel(
        _emb_body,
        out_type=jax.ShapeDtypeStruct((B, 2 * D), jnp.float32),
        mesh=plsc.VectorSubcoreMesh(
            core_axis_name="c", subcore_axis_name="s",
            num_cores=NC, num_subcores=NS,
        ),
        scratch_types=[
            pltpu.VMEM((NPW,), jnp.int32),
            pltpu.VMEM((CH, D), jnp.float32),
            pltpu.VMEM((CH, D), jnp.float32),
            pltpu.VMEM((CH, D), jnp.float32),
            pltpu.SemaphoreType.DMA,
            pltpu.SemaphoreType.DMA,
            pltpu.SemaphoreType.DMA,
            pltpu.SemaphoreType.DMA,
            pltpu.SemaphoreType.DMA,
            pltpu.SemaphoreType.DMA,
        ],
        compiler_params=pltpu.CompilerParams(use_tc_tiling_on_sc=False),
    )
    out = run(idx, table)
    return out[:, :D].reshape(tgt.shape + (D,))


# speedup vs baseline: 1.3652x; 1.0024x over previous
import jax
import jax.numpy as jnp
from jax import lax
from jax.experimental import pallas as pl
from jax.experimental.pallas import tpu as pltpu
from jax.experimental.pallas import tpu_sc as plsc

NUM_EMB = 1000000
D = 64
NI = 4096
NJ = 200
NC, NS, L = 2, 16, 16
NW = NC * NS
SUB = 128
CH = 512
NSUB = CH // SUB
NIB = NI // CH
NT = (NJ * NIB) // NW
NBUF = 3


def _emb_body(idx_hbm, table_hbm, out_hbm,
              ibuf0, ibuf1, ibuf2, rows0, rows1, rows2,
              isem0, isem1, isem2, gsem0, gsem1, gsem2,
              osem0, osem1, osem2):
    wid = lax.axis_index("s") * NC + lax.axis_index("c")
    ib = wid % NIB
    j0 = wid // NIB
    ioff = ib * CH
    ibuf = (ibuf0, ibuf1, ibuf2)
    rows = (rows0, rows1, rows2)
    isem = (isem0, isem1, isem2)
    gsem = (gsem0, gsem1, gsem2)
    osem = (osem0, osem1, osem2)

    def jpos(n):
        return j0 + 4 * n

    def idx_desc(n, b):
        return pltpu.make_async_copy(
            idx_hbm.at[jpos(n), pl.ds(ioff, CH)], ibuf[b], isem[b])

    def remap(b):
        for q in range(CH // L):
            v = ibuf[b][pl.ds(q * L, L)]
            ibuf[b][pl.ds(q * L, L)] = jnp.where(v < 0, v + (NUM_EMB + 1), v)

    def gather_descs(b):
        return [
            pltpu.make_async_copy(
                table_hbm.at[ibuf[b].at[pl.ds(s * SUB, SUB)]],
                rows[b].at[pl.ds(s * SUB, SUB)],
                gsem[b],
            )
            for s in range(NSUB)
        ]

    def write_desc(n, b):
        return pltpu.make_async_copy(
            rows[b], out_hbm.at[pl.ds(ioff, CH), jpos(n), pl.ds(0, D)],
            osem[b])

    def stage_gather(n, b):
        idx_desc(n, b).wait()
        remap(b)
        for d in gather_descs(b):
            d.start()

    idx_desc(0, 0).start()
    idx_desc(1, 1).start()
    stage_gather(0, 0)
    for j in (0, 1):
        b = j % NBUF
        stage_gather(j + 1, (b + 1) % NBUF)
        idx_desc(j + 2, (b + 2) % NBUF).start()
        for d in gather_descs(b):
            d.wait()
        write_desc(j, b).start()

    def step(k, carry):
        for m in range(NBUF):
            j = k * NBUF + 2 + m
            b = (2 + m) % NBUF
            write_desc(j - 2, (b + 1) % NBUF).wait()
            stage_gather(j + 1, (b + 1) % NBUF)
            idx_desc(j + 2, (b + 2) % NBUF).start()
            for d in gather_descs(b):
                d.wait()
            write_desc(j, b).start()
        return carry

    lax.fori_loop(0, (NT - 5) // NBUF, step, 0)

    j = NT - 3
    b = j % NBUF
    write_desc(j - 2, (b + 1) % NBUF).wait()
    stage_gather(j + 1, (b + 1) % NBUF)
    idx_desc(j + 2, (b + 2) % NBUF).start()
    for d in gather_descs(b):
        d.wait()
    write_desc(j, b).start()

    j = NT - 2
    b = j % NBUF
    write_desc(j - 2, (b + 1) % NBUF).wait()
    stage_gather(j + 1, (b + 1) % NBUF)
    for d in gather_descs(b):
        d.wait()
    write_desc(j, b).start()

    j = NT - 1
    b = j % NBUF
    write_desc(j - 2, (b + 1) % NBUF).wait()
    for d in gather_descs(b):
        d.wait()
    write_desc(j, b).start()

    for j in (NT - 2, NT - 1):
        write_desc(j, j % NBUF).wait()


@jax.jit
def kernel(tgt, table):
    idx2 = jnp.swapaxes(tgt, 0, 1).astype(jnp.int32)
    run = pl.kernel(
        _emb_body,
        out_type=jax.ShapeDtypeStruct((NI, NJ, 2 * D), jnp.float32),
        mesh=plsc.VectorSubcoreMesh(
            core_axis_name="c", subcore_axis_name="s",
            num_cores=NC, num_subcores=NS,
        ),
        scratch_types=[
            pltpu.VMEM((CH,), jnp.int32),
            pltpu.VMEM((CH,), jnp.int32),
            pltpu.VMEM((CH,), jnp.int32),
            pltpu.VMEM((CH, D), jnp.float32),
            pltpu.VMEM((CH, D), jnp.float32),
            pltpu.VMEM((CH, D), jnp.float32),
            pltpu.SemaphoreType.DMA,
            pltpu.SemaphoreType.DMA,
            pltpu.SemaphoreType.DMA,
            pltpu.SemaphoreType.DMA,
            pltpu.SemaphoreType.DMA,
            pltpu.SemaphoreType.DMA,
            pltpu.SemaphoreType.DMA,
            pltpu.SemaphoreType.DMA,
            pltpu.SemaphoreType.DMA,
        ],
        compiler_params=pltpu.CompilerParams(use_tc_tiling_on_sc=False),
    )
    out3 = run(idx2, table)
    return out3[:, :, :D]


# speedup vs baseline: 1.4702x; 1.0769x over previous
import jax
import jax.numpy as jnp
from jax import lax
from jax.experimental import pallas as pl
from jax.experimental.pallas import tpu as pltpu
from jax.experimental.pallas import tpu_sc as plsc

NUM_EMB = 1000000
D = 64
NI = 4096
NJ = 200
NC, NS, L = 2, 16, 16
NW = NC * NS
SUB = 128
CH = 512
NSUB = CH // SUB
NIB = NI // CH
NT = (NJ * NIB) // NW
NBUF = 3


def _emb_body(idx_hbm, table_hbm, out_hbm,
              ibuf0, ibuf1, ibuf2, rows0, rows1, rows2,
              isem0, isem1, isem2, gsem0, gsem1, gsem2,
              osem0, osem1, osem2):
    wid = lax.axis_index("s") * NC + lax.axis_index("c")
    ib = wid % NIB
    j0 = wid // NIB
    ioff = ib * CH
    ibuf = (ibuf0, ibuf1, ibuf2)
    rows = (rows0, rows1, rows2)
    isem = (isem0, isem1, isem2)
    gsem = (gsem0, gsem1, gsem2)
    osem = (osem0, osem1, osem2)

    def jpos(n):
        return j0 + 4 * n

    def idx_desc(n, b):
        return pltpu.make_async_copy(
            idx_hbm.at[jpos(n), pl.ds(ioff, CH)], ibuf[b], isem[b])

    def remap(b):
        for q in range(CH // L):
            v = ibuf[b][pl.ds(q * L, L)]
            ibuf[b][pl.ds(q * L, L)] = jnp.where(v < 0, v + (NUM_EMB + 1), v) * 2

    def gather_descs(b):
        return [
            pltpu.make_async_copy(
                table_hbm.at[ibuf[b].at[pl.ds(s * SUB, SUB)]],
                rows[b].at[pl.ds(s * SUB, SUB)],
                gsem[b],
            )
            for s in range(NSUB)
        ]

    def write_desc(n, b):
        return pltpu.make_async_copy(
            rows[b], out_hbm.at[pl.ds(ioff, CH), jpos(n), pl.ds(0, D)],
            osem[b])

    def stage_gather(n, b):
        idx_desc(n, b).wait()
        remap(b)
        for d in gather_descs(b):
            d.start()

    idx_desc(0, 0).start()
    idx_desc(1, 1).start()
    stage_gather(0, 0)
    for j in (0, 1):
        b = j % NBUF
        stage_gather(j + 1, (b + 1) % NBUF)
        idx_desc(j + 2, (b + 2) % NBUF).start()
        for d in gather_descs(b):
            d.wait()
        write_desc(j, b).start()

    def step(k, carry):
        for m in range(NBUF):
            j = k * NBUF + 2 + m
            b = (2 + m) % NBUF
            write_desc(j - 2, (b + 1) % NBUF).wait()
            stage_gather(j + 1, (b + 1) % NBUF)
            idx_desc(j + 2, (b + 2) % NBUF).start()
            for d in gather_descs(b):
                d.wait()
            write_desc(j, b).start()
        return carry

    lax.fori_loop(0, (NT - 5) // NBUF, step, 0)

    j = NT - 3
    b = j % NBUF
    write_desc(j - 2, (b + 1) % NBUF).wait()
    stage_gather(j + 1, (b + 1) % NBUF)
    idx_desc(j + 2, (b + 2) % NBUF).start()
    for d in gather_descs(b):
        d.wait()
    write_desc(j, b).start()

    j = NT - 2
    b = j % NBUF
    write_desc(j - 2, (b + 1) % NBUF).wait()
    stage_gather(j + 1, (b + 1) % NBUF)
    for d in gather_descs(b):
        d.wait()
    write_desc(j, b).start()

    j = NT - 1
    b = j % NBUF
    write_desc(j - 2, (b + 1) % NBUF).wait()
    for d in gather_descs(b):
        d.wait()
    write_desc(j, b).start()

    for j in (NT - 2, NT - 1):
        write_desc(j, j % NBUF).wait()


@jax.jit
def kernel(tgt, table):
    idx2 = jnp.swapaxes(tgt, 0, 1).astype(jnp.int32)
    table_v = jnp.pad(table, ((0, 0), (0, D))).reshape(2 * (NUM_EMB + 1), D)
    run = pl.kernel(
        _emb_body,
        out_type=jax.ShapeDtypeStruct((NI, NJ, 2 * D), jnp.float32),
        mesh=plsc.VectorSubcoreMesh(
            core_axis_name="c", subcore_axis_name="s",
            num_cores=NC, num_subcores=NS,
        ),
        scratch_types=[
            pltpu.VMEM((CH,), jnp.int32),
            pltpu.VMEM((CH,), jnp.int32),
            pltpu.VMEM((CH,), jnp.int32),
            pltpu.VMEM((CH, D), jnp.float32),
            pltpu.VMEM((CH, D), jnp.float32),
            pltpu.VMEM((CH, D), jnp.float32),
            pltpu.SemaphoreType.DMA,
            pltpu.SemaphoreType.DMA,
            pltpu.SemaphoreType.DMA,
            pltpu.SemaphoreType.DMA,
            pltpu.SemaphoreType.DMA,
            pltpu.SemaphoreType.DMA,
            pltpu.SemaphoreType.DMA,
            pltpu.SemaphoreType.DMA,
            pltpu.SemaphoreType.DMA,
        ],
        compiler_params=pltpu.CompilerParams(use_tc_tiling_on_sc=False),
    )
    out3 = run(idx2, table_v)
    return out3[:, :, :D]
